# baseline (speedup 1.0000x reference)
import functools

import jax
import jax.numpy as jnp
from jax import lax
from jax.experimental import pallas as pl
from jax.experimental.pallas import tpu as pltpu
from jax.experimental.pallas import tpu_sc as plsc

B = 16384
V = 1000000
D = 64
NEG = 20
NC = 2
NS = 16
NW = NC * NS
RPW = B // NW
CHUNK = 128
NCHUNK = RPW // CHUNK
TBLK = 32768
NTB = (V + TBLK - 1) // TBLK
VPAD = NTB * TBLK


def _widen_table(emb):
    def body(x_ref, o_ref):
        t = x_ref[...].T
        o_ref[:, pl.ds(0, D)] = lax.slice(t, (0, 0), (TBLK // 2, D))
        o_ref[:, pl.ds(D, D)] = lax.slice(t, (TBLK // 2, 0), (TBLK, D))

    return pl.pallas_call(
        body,
        grid=(NTB,),
        in_specs=[pl.BlockSpec((D, TBLK), lambda g: (0, g))],
        out_specs=pl.BlockSpec((TBLK // 2, 128), lambda g: (g, 0)),
        out_shape=jax.ShapeDtypeStruct((VPAD // 2, 128), jnp.float32),
    )(emb.T)


def _sc_scores(ew, inp2d, pos2d, negT2d):
    mesh = plsc.VectorSubcoreMesh(core_axis_name="c", subcore_axis_name="s")

    @functools.partial(
        pl.kernel,
        mesh=mesh,
        compiler_params=pltpu.CompilerParams(
            needs_layout_passes=False, use_tc_tiling_on_sc=True),
        out_type=(
            jax.ShapeDtypeStruct((B,), jnp.float32),
            jax.ShapeDtypeStruct((B * NEG,), jnp.float32),
        ),
        scratch_types=[
            pltpu.VMEM((NCHUNK, 128), jnp.int32),
            pltpu.VMEM((NCHUNK, 128), jnp.int32),
            pltpu.VMEM((NEG * NCHUNK, 128), jnp.int32),
            pltpu.VMEM((CHUNK, 128), jnp.float32),
            pltpu.VMEM((CHUNK, 128), jnp.float32),
            pltpu.VMEM((CHUNK, 128), jnp.float32),
            pltpu.VMEM((CHUNK, 128), jnp.float32),
            pltpu.VMEM((CHUNK, 128), jnp.float32),
            pltpu.VMEM((CHUNK, 128), jnp.float32),
            pltpu.VMEM((CHUNK, 128), jnp.float32),
            pltpu.VMEM((8, 128), jnp.int32),
            pltpu.VMEM((CHUNK,), jnp.float32),
            pltpu.VMEM((CHUNK * NEG,), jnp.float32),
            pltpu.SemaphoreType.DMA,
            pltpu.SemaphoreType.DMA,
            pltpu.SemaphoreType.DMA,
            pltpu.SemaphoreType.DMA,
            pltpu.SemaphoreType.DMA,
        ],
    )
    def k(ew_h, inp_h, pos_h, neg_h, pos_out, neg_out,
          aidx, pidx, nidx, waA, waB, wpA, wpB, wnA, wnB, wnC,
          phys, postage, negstage, semA, semP, semN0, semN1, semN2):
        wid = lax.axis_index("s") * NC + lax.axis_index("c")
        lane = lax.iota(jnp.int32, 16)
        wa = (waA, waB)
        wp = (wpA, wpB)
        wn = (wnA, wnB, wnC)
        semN = (semN0, semN1, semN2)

        pltpu.sync_copy(inp_h.at[pl.ds(wid * NCHUNK, NCHUNK)], aidx)
        pltpu.sync_copy(pos_h.at[pl.ds(wid * NCHUNK, NCHUNK)], pidx)
        for n in range(NEG):
            pltpu.sync_copy(
                neg_h.at[pl.ds(n * (B // 128) + wid * NCHUNK, NCHUNK)],
                nidx.at[pl.ds(n * NCHUNK, NCHUNK)])

        HB = TBLK // 2
        def to_phys(src_ref, src_row, prow):
            for k in range(8):
                v = src_ref[src_row, pl.ds(k * 16, 16)]
                phys[prow, pl.ds(k * 16, 16)] = (
                    lax.shift_left(lax.shift_right_logical(v, 15), 14)
                    | (v & (HB - 1)))

        def fire_ap(c):
            to_phys(aidx, c, 3 + (c % 2))
            to_phys(pidx, c, 5 + (c % 2))
            pltpu.async_copy(ew_h.at[phys.at[3 + (c % 2)]], wa[c % 2], semA)
            pltpu.async_copy(ew_h.at[phys.at[5 + (c % 2)]], wp[c % 2], semP)

        def fire_n(c, n):
            bi = (c * NEG + n) % 3
            to_phys(nidx, n * NCHUNK + c, bi)
            pltpu.async_copy(ew_h.at[phys.at[bi]], wn[bi], semN[bi])

        def drain(buf, sem):
            pltpu.make_async_copy(ew_h.at[pl.ds(0, CHUNK)], buf, sem).wait()

        fire_ap(0)
        fire_n(0, 0)
        fire_n(0, 1)

        def dots(bufx, bufy, xrow, yrow, stage, soff):
            def group(g, carry):
                rowv = lane + g * 16
                hx = (aidx[xrow, pl.ds(g * 16, 16)] >> 8) & D
                hy_src, hy_row = yrow
                hy = (hy_src[hy_row, pl.ds(g * 16, 16)] >> 8) & D

                def dbody(d, acc):
                    rot = (lane + d) & (D - 1)
                    x_d = plsc.load_gather(bufx, [rowv, hx + rot])
                    y_d = plsc.load_gather(bufy, [rowv, hy + rot])
                    return acc + x_d * y_d

                acc = lax.fori_loop(0, D, dbody,
                                    jnp.zeros((16,), jnp.float32))
                stage[pl.ds(soff + g * 16, 16)] = acc
                return carry

            lax.fori_loop(0, CHUNK // 16, group, 0)

        for c in range(NCHUNK):
            gc = wid * NCHUNK + c
            drain(wa[c % 2], semA)
            drain(wp[c % 2], semP)
            if c + 1 < NCHUNK:
                fire_ap(c + 1)
            dots(wa[c % 2], wp[c % 2], c, (pidx, c), postage, 0)
            pltpu.sync_copy(postage, pos_out.at[pl.ds(gc * CHUNK, CHUNK)])
            for n in range(NEG):
                bi = (c * NEG + n) % 3
                drain(wn[bi], semN[bi])
                nf = n + 2
                if nf < NEG:
                    fire_n(c, nf)
                elif c + 1 < NCHUNK:
                    fire_n(c + 1, nf - NEG)
                dots(wa[c % 2], wn[bi], c, (nidx, n * NCHUNK + c),
                     negstage, n * CHUNK)
            pltpu.sync_copy(
                negstage,
                neg_out.at[pl.ds(gc * CHUNK * NEG, CHUNK * NEG)])

    return k(ew, inp2d, pos2d, negT2d)


def _tc_loss(pos_s, neg_s):
    def body(p_ref, n_ref, o_ref):
        p = jax.nn.log_sigmoid(p_ref[...])
        n = jax.nn.log_sigmoid(n_ref[...])
        o_ref[...] = (jnp.sum(n) - jnp.sum(p))[None, None]

    out = pl.pallas_call(
        body,
        out_shape=jax.ShapeDtypeStruct((1, 1), jnp.float32),
    )(pos_s.reshape(B // 128, 128), neg_s.reshape(B * NEG // 128, 128))
    return out[0, 0]


def kernel(input, pos_input, neg_input, Embedding):
    inp = input.astype(jnp.int32).reshape(B // 128, 128)
    pos = pos_input.astype(jnp.int32).reshape(B // 128, 128)
    negT = neg_input.astype(jnp.int32).T.reshape(NEG * B // 128, 128)
    ew = _widen_table(Embedding)
    pos_s, neg_s = _sc_scores(ew, inp, pos, negT)
    return _tc_loss(pos_s, neg_s)
